# baseline (speedup 1.0000x reference)
import jax
import jax.numpy as jnp
from jax.experimental import pallas as pl
from jax.experimental.pallas import tpu as pltpu


_B, _NS, _D = 8, 2048, 128
_BQ = 512
_CK = 512


def _nn_kernel(x_ref, y_ref, md_ref, idx_ref, yn_ref):
    q = pl.program_id(1)
    y = y_ref[0]

    @pl.when(q == 0)
    def _fill_yn():
        yn_ref[...] = jnp.sum(y * y, axis=1, keepdims=True)

    x = x_ref[0]
    xm2 = x * -2.0
    m = None
    idx = None
    for c in range(_NS // _CK):
        yc = y[c * _CK:(c + 1) * _CK, :]
        g = jax.lax.dot_general(
            yc, xm2,
            dimension_numbers=(((1,), (1,)), ((), ())),
            precision=jax.lax.Precision.HIGHEST,
            preferred_element_type=jnp.float32,
        )
        d2 = g + yn_ref[c * _CK:(c + 1) * _CK, :]
        mc = jnp.min(d2, axis=0, keepdims=True)
        iota = jax.lax.broadcasted_iota(jnp.int32, d2.shape, 0)
        hit = jnp.where(d2 <= mc, iota, jnp.int32(_CK))
        ic = jnp.min(hit, axis=0) + jnp.int32(c * _CK)
        if m is None:
            m, idx = mc, ic
        else:
            better = mc < m
            idx = jnp.where(better[0], ic, idx)
            m = jnp.minimum(m, mc)
    xn = jax.lax.dot_general(
        jnp.ones((1, _D), jnp.float32), x * x,
        dimension_numbers=(((1,), (1,)), ((), ())),
        precision=jax.lax.Precision.HIGHEST,
        preferred_element_type=jnp.float32,
    )
    md_ref[0, 0, :] = jnp.sqrt(jnp.maximum(xn[0] + m[0], 0.0))
    idx_ref[0, 0, :] = idx


def kernel(noisy_data, data, condition_mask):
    x = jnp.where(condition_mask[None, None, :], data, noisy_data)
    nq = _NS // _BQ
    grid = (_B, nq)
    md, idx = pl.pallas_call(
        _nn_kernel,
        grid=grid,
        in_specs=[
            pl.BlockSpec((1, _BQ, _D), lambda b, q: (b, q, 0)),
            pl.BlockSpec((1, _NS, _D), lambda b, q: (b, 0, 0)),
        ],
        out_specs=[
            pl.BlockSpec((1, 1, _BQ), lambda b, q: (b * nq + q, 0, 0)),
            pl.BlockSpec((1, 1, _BQ), lambda b, q: (b * nq + q, 0, 0)),
        ],
        out_shape=[
            jax.ShapeDtypeStruct((_B * nq, 1, _BQ), jnp.float32),
            jax.ShapeDtypeStruct((_B * nq, 1, _BQ), jnp.int32),
        ],
        scratch_shapes=[pltpu.VMEM((_NS, 1), jnp.float32)],
    )(x, data)
    return md.reshape(_B, _NS), idx.reshape(_B, _NS)


# speedup vs baseline: 1.7755x; 1.7755x over previous
import jax
import jax.numpy as jnp
from jax.experimental import pallas as pl
from jax.experimental.pallas import tpu as pltpu


_B, _NS, _D = 8, 2048, 128
_BQ = 512


def _nn_kernel(x_ref, y_ref, md_ref, idx_ref, yn_ref, yc_ref):
    q = pl.program_id(1)

    @pl.when(q == 0)
    def _fill_scratch():
        y = y_ref[0]
        yn_ref[...] = jnp.sum(y * y, axis=1, keepdims=True)
        yh = y.astype(jnp.bfloat16)
        yl = (y - yh.astype(jnp.float32)).astype(jnp.bfloat16)
        yc_ref[...] = jnp.concatenate([yh, yh, yl], axis=1)

    x = x_ref[0]
    xm2 = x * -2.0
    xh = xm2.astype(jnp.bfloat16)
    xl = (xm2 - xh.astype(jnp.float32)).astype(jnp.bfloat16)
    xc = jnp.concatenate([xh, xl, xh], axis=1)
    g = jax.lax.dot_general(
        yc_ref[...], xc,
        dimension_numbers=(((1,), (1,)), ((), ())),
        preferred_element_type=jnp.float32,
    )
    d2 = g + yn_ref[...]
    m = jnp.min(d2, axis=0, keepdims=True)
    iota = jax.lax.broadcasted_iota(jnp.int32, d2.shape, 0)
    hit = jnp.where(d2 <= m, iota, jnp.int32(_NS))
    idx = jnp.min(hit, axis=0)
    xn = jax.lax.dot_general(
        jnp.ones((1, _D), jnp.float32), x * x,
        dimension_numbers=(((1,), (1,)), ((), ())),
        precision=jax.lax.Precision.HIGHEST,
        preferred_element_type=jnp.float32,
    )
    md_ref[0, 0, :] = jnp.sqrt(jnp.maximum(xn[0] + m[0], 0.0))
    idx_ref[0, 0, :] = idx


def kernel(noisy_data, data, condition_mask):
    x = jnp.where(condition_mask[None, None, :], data, noisy_data)
    nq = _NS // _BQ
    grid = (_B, nq)
    md, idx = pl.pallas_call(
        _nn_kernel,
        grid=grid,
        in_specs=[
            pl.BlockSpec((1, _BQ, _D), lambda b, q: (b, q, 0)),
            pl.BlockSpec((1, _NS, _D), lambda b, q: (b, 0, 0)),
        ],
        out_specs=[
            pl.BlockSpec((1, 1, _BQ), lambda b, q: (b * nq + q, 0, 0)),
            pl.BlockSpec((1, 1, _BQ), lambda b, q: (b * nq + q, 0, 0)),
        ],
        out_shape=[
            jax.ShapeDtypeStruct((_B * nq, 1, _BQ), jnp.float32),
            jax.ShapeDtypeStruct((_B * nq, 1, _BQ), jnp.int32),
        ],
        scratch_shapes=[
            pltpu.VMEM((_NS, 1), jnp.float32),
            pltpu.VMEM((_NS, 3 * _D), jnp.bfloat16),
        ],
    )(x, data)
    return md.reshape(_B, _NS), idx.reshape(_B, _NS)
